# Initial kernel scaffold; baseline (speedup 1.0000x reference)
import jax
import jax.numpy as jnp
from jax.experimental import pallas as pl


def kernel(x_user, x_item, edge_index_user_clicks_item, edge_index_item_rev_clicks_user, W_self_u2i, W_neigh_u2i, b_u2i, W_self_i2u, W_neigh_i2u, b_i2u):
    raise NotImplementedError("write your pallas kernel here")


# speedup vs baseline: 3.7949x; 3.7949x over previous
import functools

import jax
import jax.numpy as jnp
from jax import lax
from jax.experimental import pallas as pl
from jax.experimental.pallas import tpu as pltpu
from jax.experimental.pallas import tpu_sc as plsc

N_DST = 10000
D = 128
DH = D // 2
E_EDGES = 320000
LANES = 128
N_SUB = 16
ROWS = E_EDGES // LANES
ROWS_PER_TILE = (-(-ROWS // (N_SUB * 8))) * 8
ROWS_PAD = ROWS_PER_TILE * N_SUB
N_PAD = 10112
SLICE = N_PAD // N_SUB
DEG_W = 16
DUMMY_DST = N_DST + 8


def _sc_body(xu_lo, xu_hi, xi_lo, xi_hi, src_a, dst_a, src_b, dst_b,
             zf, zd, ones_h,
             agg_a_lo, agg_a_hi, deg_a, agg_b_lo, agg_b_hi, deg_b,
             src_v, dst_v, rows_v, ones_v, agg_sp, deg_sp, sem):
  c = lax.axis_index("c")
  s = lax.axis_index("s")

  def run(x_lo, x_hi, src_h, dst_h, agg_out_lo, agg_out_hi, deg_out):
    pltpu.sync_copy(src_h.at[pl.ds(s * ROWS_PER_TILE, ROWS_PER_TILE)], src_v)
    pltpu.sync_copy(dst_h.at[pl.ds(s * ROWS_PER_TILE, ROWS_PER_TILE)], dst_v)
    pltpu.sync_copy(ones_h, ones_v)

    def one_pass(x_src, agg_out, first):
      pltpu.sync_copy(zf, agg_sp.at[pl.ds(s * SLICE, SLICE)])
      if first:
        pltpu.sync_copy(zd, deg_sp.at[pl.ds(s * SLICE, SLICE)])
      plsc.subcore_barrier()

      def step(i, carry):
        pltpu.async_copy(x_src.at[src_v.at[i]], rows_v, sem).wait()
        pltpu.sync_copy(rows_v, agg_sp.at[dst_v.at[i]], add=True)
        if first:
          pltpu.sync_copy(ones_v, deg_sp.at[dst_v.at[i]], add=True)
        return carry

      lax.fori_loop(0, ROWS_PER_TILE, step, 0)
      plsc.subcore_barrier()
      pltpu.sync_copy(agg_sp.at[pl.ds(s * SLICE, SLICE)],
                      agg_out.at[pl.ds(s * SLICE, SLICE)])
      if first:
        pltpu.sync_copy(deg_sp.at[pl.ds(s * SLICE, SLICE)],
                        deg_out.at[pl.ds(s * SLICE, SLICE)])

    one_pass(x_lo, agg_out_lo, True)
    one_pass(x_hi, agg_out_hi, False)

  @pl.when(c == 0)
  def _():
    run(xu_lo, xu_hi, src_a, dst_a, agg_a_lo, agg_a_hi, deg_a)

  @pl.when(c == 1)
  def _():
    run(xi_lo, xi_hi, src_b, dst_b, agg_b_lo, agg_b_hi, deg_b)


_sc_call = functools.partial(
    pl.kernel,
    out_type=[
        jax.ShapeDtypeStruct((N_PAD, DH), jnp.float32),
        jax.ShapeDtypeStruct((N_PAD, DH), jnp.float32),
        jax.ShapeDtypeStruct((N_PAD, DEG_W), jnp.float32),
        jax.ShapeDtypeStruct((N_PAD, DH), jnp.float32),
        jax.ShapeDtypeStruct((N_PAD, DH), jnp.float32),
        jax.ShapeDtypeStruct((N_PAD, DEG_W), jnp.float32),
    ],
    mesh=plsc.VectorSubcoreMesh(core_axis_name="c", subcore_axis_name="s"),
    compiler_params=pltpu.CompilerParams(use_tc_tiling_on_sc=False),
    scratch_types=[
        pltpu.VMEM((ROWS_PER_TILE, LANES), jnp.int32),
        pltpu.VMEM((ROWS_PER_TILE, LANES), jnp.int32),
        pltpu.VMEM((LANES, DH), jnp.float32),
        pltpu.VMEM((LANES, DEG_W), jnp.float32),
        pltpu.VMEM_SHARED((N_PAD, DH), jnp.float32),
        pltpu.VMEM_SHARED((N_PAD, DEG_W), jnp.float32),
        pltpu.SemaphoreType.DMA,
    ],
)(_sc_body)


def _tc_body(x_i, agg_i_lo, agg_i_hi, deg_i, x_u, agg_u_lo, agg_u_hi, deg_u,
             ws_a, wn_a, b_a, ws_b, wn_b, b_b, out_i, out_u):
  def sage(x, agg_lo, agg_hi, deg, ws, wn, b):
    d = jnp.max(deg[...], axis=1, keepdims=True)
    agg = jnp.concatenate([agg_lo[...], agg_hi[...]], axis=1)
    mean = agg / jnp.maximum(d, 1.0)
    return (jnp.dot(x[...], ws[...], preferred_element_type=jnp.float32)
            + jnp.dot(mean, wn[...], preferred_element_type=jnp.float32)
            + b[...])

  out_i[...] = sage(x_i, agg_i_lo, agg_i_hi, deg_i, ws_a, wn_a, b_a)
  out_u[...] = sage(x_u, agg_u_lo, agg_u_hi, deg_u, ws_b, wn_b, b_b)


_TC_BLK = 1000


def _tc_call(x_i, agg_i_lo, agg_i_hi, deg_i, x_u, agg_u_lo, agg_u_hi, deg_u,
             ws_a, wn_a, b_a, ws_b, wn_b, b_b):
  row = lambda i: (i, 0)
  fix = lambda i: (0, 0)
  return pl.pallas_call(
      _tc_body,
      grid=(N_DST // _TC_BLK,),
      in_specs=[
          pl.BlockSpec((_TC_BLK, D), row),
          pl.BlockSpec((_TC_BLK, DH), row),
          pl.BlockSpec((_TC_BLK, DH), row),
          pl.BlockSpec((_TC_BLK, DEG_W), row),
          pl.BlockSpec((_TC_BLK, D), row),
          pl.BlockSpec((_TC_BLK, DH), row),
          pl.BlockSpec((_TC_BLK, DH), row),
          pl.BlockSpec((_TC_BLK, DEG_W), row),
          pl.BlockSpec((D, D), fix),
          pl.BlockSpec((D, D), fix),
          pl.BlockSpec((1, D), fix),
          pl.BlockSpec((D, D), fix),
          pl.BlockSpec((D, D), fix),
          pl.BlockSpec((1, D), fix),
      ],
      out_specs=[pl.BlockSpec((_TC_BLK, D), row),
                 pl.BlockSpec((_TC_BLK, D), row)],
      out_shape=[jax.ShapeDtypeStruct((N_DST, D), jnp.float32)] * 2,
  )(x_i, agg_i_lo, agg_i_hi, deg_i, x_u, agg_u_lo, agg_u_hi, deg_u,
    ws_a, wn_a, b_a, ws_b, wn_b, b_b)


def _pad_edges(ei):
  n_pad = ROWS_PAD * LANES - E_EDGES
  src = jnp.concatenate(
      [ei[0].astype(jnp.int32), jnp.zeros((n_pad,), jnp.int32)])
  dst = jnp.concatenate(
      [ei[1].astype(jnp.int32), jnp.full((n_pad,), DUMMY_DST, jnp.int32)])
  return src.reshape(ROWS_PAD, LANES), dst.reshape(ROWS_PAD, LANES)


def kernel(x_user, x_item, edge_index_user_clicks_item,
           edge_index_item_rev_clicks_user, W_self_u2i, W_neigh_u2i, b_u2i,
           W_self_i2u, W_neigh_i2u, b_i2u):
  src_a, dst_a = _pad_edges(edge_index_user_clicks_item)
  src_b, dst_b = _pad_edges(edge_index_item_rev_clicks_user)
  zf = jnp.zeros((SLICE, DH), jnp.float32)
  zd = jnp.zeros((SLICE, DEG_W), jnp.float32)
  ones_h = jnp.ones((LANES, DEG_W), jnp.float32)
  agg_i_lo, agg_i_hi, deg_i, agg_u_lo, agg_u_hi, deg_u = _sc_call(
      x_user[:, :DH], x_user[:, DH:], x_item[:, :DH], x_item[:, DH:],
      src_a, dst_a, src_b, dst_b, zf, zd, ones_h)
  out_item, out_user = _tc_call(
      x_item, agg_i_lo[:N_DST], agg_i_hi[:N_DST], deg_i[:N_DST],
      x_user, agg_u_lo[:N_DST], agg_u_hi[:N_DST], deg_u[:N_DST],
      W_self_u2i, W_neigh_u2i, b_u2i.reshape(1, D),
      W_self_i2u, W_neigh_i2u, b_i2u.reshape(1, D))
  return (out_item, out_user)


# speedup vs baseline: 4.9127x; 1.2946x over previous
import functools

import jax
import jax.numpy as jnp
from jax import lax
from jax.experimental import pallas as pl
from jax.experimental.pallas import tpu as pltpu
from jax.experimental.pallas import tpu_sc as plsc

N_DST = 10000
D = 128
DH = D // 2
E_EDGES = 320000
LANES = 128
N_SUB = 16
ROWS = E_EDGES // LANES
ROWS_PER_TILE = (-(-ROWS // (N_SUB * 8))) * 8
ROWS_PAD = ROWS_PER_TILE * N_SUB
N_PAD = 10112
SLICE = N_PAD // N_SUB
DEG_W = 16
DUMMY_DST = N_DST + 8


GRP = 2
STEP = 2 * GRP
K_ITERS = ROWS_PER_TILE // STEP


def _sc_body(xu_lo, xu_hi, xi_lo, xi_hi, src_a, dst_a, src_b, dst_b,
             zf, zd, ones_h,
             agg_a_lo, agg_a_hi, deg_a, agg_b_lo, agg_b_hi, deg_b,
             src_v, dst_v, rows_v, ones_v, agg_sp, deg_sp,
             gsem_a, gsem_b, ssem_a, ssem_b):
  c = lax.axis_index("c")
  s = lax.axis_index("s")
  gsem = (gsem_a, gsem_b)
  ssem = (ssem_a, ssem_b)

  def run(x_lo, x_hi, src_h, dst_h, agg_out_lo, agg_out_hi, deg_out):
    pltpu.sync_copy(src_h.at[pl.ds(s * ROWS_PER_TILE, ROWS_PER_TILE)], src_v)
    pltpu.sync_copy(dst_h.at[pl.ds(s * ROWS_PER_TILE, ROWS_PER_TILE)], dst_v)
    pltpu.sync_copy(ones_h, ones_v)

    def one_pass(x_src, agg_out, first):
      pltpu.sync_copy(zf, agg_sp.at[pl.ds(s * SLICE, SLICE)])
      if first:
        pltpu.sync_copy(zd, deg_sp.at[pl.ds(s * SLICE, SLICE)])
      plsc.subcore_barrier()

      def g_fire(side, row0, clamp=False):
        for b in range(GRP):
          r = jnp.minimum(row0 + b, ROWS_PER_TILE - 1) if clamp else row0 + b
          pltpu.async_copy(x_src.at[src_v.at[r]], rows_v.at[side * GRP + b],
                           gsem[side])

      def g_drain(side):
        for b in range(GRP):
          pltpu.make_async_copy(x_src.at[src_v.at[0]],
                                rows_v.at[side * GRP + b], gsem[side]).wait()

      def s_fire(side, row0):
        for b in range(GRP):
          r = row0 + b
          pltpu.async_copy(rows_v.at[side * GRP + b],
                           agg_sp.at[dst_v.at[r]], ssem[side], add=True)
          if first:
            pltpu.async_copy(ones_v, deg_sp.at[dst_v.at[r]], ssem[side],
                             add=True)

      def s_drain(side):
        for b in range(GRP):
          pltpu.make_async_copy(rows_v.at[side * GRP + b],
                                agg_sp.at[dst_v.at[0]], ssem[side]).wait()
          if first:
            pltpu.make_async_copy(ones_v, deg_sp.at[dst_v.at[0]],
                                  ssem[side]).wait()

      g_fire(0, 0)

      def step(j, carry):
        r0 = j * STEP
        g_fire(1, r0 + GRP)
        g_drain(0)
        s_fire(0, r0)
        s_drain(0)
        g_fire(0, r0 + STEP, clamp=True)
        g_drain(1)
        s_fire(1, r0 + GRP)
        s_drain(1)
        return carry

      lax.fori_loop(0, K_ITERS, step, 0)
      g_drain(0)
      plsc.subcore_barrier()
      pltpu.sync_copy(agg_sp.at[pl.ds(s * SLICE, SLICE)],
                      agg_out.at[pl.ds(s * SLICE, SLICE)])
      if first:
        pltpu.sync_copy(deg_sp.at[pl.ds(s * SLICE, SLICE)],
                        deg_out.at[pl.ds(s * SLICE, SLICE)])

    one_pass(x_lo, agg_out_lo, True)
    one_pass(x_hi, agg_out_hi, False)

  @pl.when(c == 0)
  def _():
    run(xu_lo, xu_hi, src_a, dst_a, agg_a_lo, agg_a_hi, deg_a)

  @pl.when(c == 1)
  def _():
    run(xi_lo, xi_hi, src_b, dst_b, agg_b_lo, agg_b_hi, deg_b)


_sc_call = functools.partial(
    pl.kernel,
    out_type=[
        jax.ShapeDtypeStruct((N_PAD, DH), jnp.float32),
        jax.ShapeDtypeStruct((N_PAD, DH), jnp.float32),
        jax.ShapeDtypeStruct((N_PAD, DEG_W), jnp.float32),
        jax.ShapeDtypeStruct((N_PAD, DH), jnp.float32),
        jax.ShapeDtypeStruct((N_PAD, DH), jnp.float32),
        jax.ShapeDtypeStruct((N_PAD, DEG_W), jnp.float32),
    ],
    mesh=plsc.VectorSubcoreMesh(core_axis_name="c", subcore_axis_name="s"),
    compiler_params=pltpu.CompilerParams(use_tc_tiling_on_sc=False),
    scratch_types=[
        pltpu.VMEM((ROWS_PER_TILE, LANES), jnp.int32),
        pltpu.VMEM((ROWS_PER_TILE, LANES), jnp.int32),
        pltpu.VMEM((2 * GRP, LANES, DH), jnp.float32),
        pltpu.VMEM((LANES, DEG_W), jnp.float32),
        pltpu.VMEM_SHARED((N_PAD, DH), jnp.float32),
        pltpu.VMEM_SHARED((N_PAD, DEG_W), jnp.float32),
        pltpu.SemaphoreType.DMA,
        pltpu.SemaphoreType.DMA,
        pltpu.SemaphoreType.DMA,
        pltpu.SemaphoreType.DMA,
    ],
)(_sc_body)


def _tc_body(x_i, agg_i_lo, agg_i_hi, deg_i, x_u, agg_u_lo, agg_u_hi, deg_u,
             ws_a, wn_a, b_a, ws_b, wn_b, b_b, out_i, out_u):
  def sage(x, agg_lo, agg_hi, deg, ws, wn, b):
    d = jnp.max(deg[...], axis=1, keepdims=True)
    agg = jnp.concatenate([agg_lo[...], agg_hi[...]], axis=1)
    mean = agg / jnp.maximum(d, 1.0)
    return (jnp.dot(x[...], ws[...], preferred_element_type=jnp.float32)
            + jnp.dot(mean, wn[...], preferred_element_type=jnp.float32)
            + b[...])

  out_i[...] = sage(x_i, agg_i_lo, agg_i_hi, deg_i, ws_a, wn_a, b_a)
  out_u[...] = sage(x_u, agg_u_lo, agg_u_hi, deg_u, ws_b, wn_b, b_b)


_TC_BLK = 1000


def _tc_call(x_i, agg_i_lo, agg_i_hi, deg_i, x_u, agg_u_lo, agg_u_hi, deg_u,
             ws_a, wn_a, b_a, ws_b, wn_b, b_b):
  row = lambda i: (i, 0)
  fix = lambda i: (0, 0)
  return pl.pallas_call(
      _tc_body,
      grid=(N_DST // _TC_BLK,),
      in_specs=[
          pl.BlockSpec((_TC_BLK, D), row),
          pl.BlockSpec((_TC_BLK, DH), row),
          pl.BlockSpec((_TC_BLK, DH), row),
          pl.BlockSpec((_TC_BLK, DEG_W), row),
          pl.BlockSpec((_TC_BLK, D), row),
          pl.BlockSpec((_TC_BLK, DH), row),
          pl.BlockSpec((_TC_BLK, DH), row),
          pl.BlockSpec((_TC_BLK, DEG_W), row),
          pl.BlockSpec((D, D), fix),
          pl.BlockSpec((D, D), fix),
          pl.BlockSpec((1, D), fix),
          pl.BlockSpec((D, D), fix),
          pl.BlockSpec((D, D), fix),
          pl.BlockSpec((1, D), fix),
      ],
      out_specs=[pl.BlockSpec((_TC_BLK, D), row),
                 pl.BlockSpec((_TC_BLK, D), row)],
      out_shape=[jax.ShapeDtypeStruct((N_DST, D), jnp.float32)] * 2,
  )(x_i, agg_i_lo, agg_i_hi, deg_i, x_u, agg_u_lo, agg_u_hi, deg_u,
    ws_a, wn_a, b_a, ws_b, wn_b, b_b)


def _pad_edges(ei):
  n_pad = ROWS_PAD * LANES - E_EDGES
  src = jnp.concatenate(
      [ei[0].astype(jnp.int32), jnp.zeros((n_pad,), jnp.int32)])
  dst = jnp.concatenate(
      [ei[1].astype(jnp.int32), jnp.full((n_pad,), DUMMY_DST, jnp.int32)])
  return src.reshape(ROWS_PAD, LANES), dst.reshape(ROWS_PAD, LANES)


def kernel(x_user, x_item, edge_index_user_clicks_item,
           edge_index_item_rev_clicks_user, W_self_u2i, W_neigh_u2i, b_u2i,
           W_self_i2u, W_neigh_i2u, b_i2u):
  src_a, dst_a = _pad_edges(edge_index_user_clicks_item)
  src_b, dst_b = _pad_edges(edge_index_item_rev_clicks_user)
  zf = jnp.zeros((SLICE, DH), jnp.float32)
  zd = jnp.zeros((SLICE, DEG_W), jnp.float32)
  ones_h = jnp.ones((LANES, DEG_W), jnp.float32)
  agg_i_lo, agg_i_hi, deg_i, agg_u_lo, agg_u_hi, deg_u = _sc_call(
      x_user[:, :DH], x_user[:, DH:], x_item[:, :DH], x_item[:, DH:],
      src_a, dst_a, src_b, dst_b, zf, zd, ones_h)
  out_item, out_user = _tc_call(
      x_item, agg_i_lo[:N_DST], agg_i_hi[:N_DST], deg_i[:N_DST],
      x_user, agg_u_lo[:N_DST], agg_u_hi[:N_DST], deg_u[:N_DST],
      W_self_u2i, W_neigh_u2i, b_u2i.reshape(1, D),
      W_self_i2u, W_neigh_i2u, b_i2u.reshape(1, D))
  return (out_item, out_user)
